# Initial kernel scaffold; baseline (speedup 1.0000x reference)
import jax
import jax.numpy as jnp
from jax.experimental import pallas as pl


def kernel(h, e, edge_index, params):
    raise NotImplementedError("write your pallas kernel here")


# speedup vs baseline: 1.0928x; 1.0928x over previous
import jax
import jax.numpy as jnp
from jax.experimental import pallas as pl


def _copy_body(x_ref, o_ref):
    o_ref[...] = x_ref[...]


def _gat(h_in, src, dst, W, al, ar, H, d):
    Nn = h_in.shape[0]
    z = (h_in @ W).reshape(Nn, H, d)
    el = jnp.sum(z * al[None], axis=-1)
    er = jnp.sum(z * ar[None], axis=-1)
    logits = jax.nn.leaky_relu(el[src] + er[dst], 0.2)
    ex = jnp.exp(logits)
    den = jax.ops.segment_sum(ex, dst, num_segments=Nn) + 1e-9
    num = jax.ops.segment_sum(z[src] * ex[:, :, None], dst, num_segments=Nn)
    out = (num / den[:, :, None]).reshape(Nn, H * d)
    out = jax.nn.elu(out)
    out = out + h_in
    return out


def kernel(h, e, edge_index, params):
    p = params
    src = edge_index[0]
    dst = edge_index[1]
    x = h @ p["W_emb"] + p["b_emb"]
    x = pl.pallas_call(
        _copy_body, out_shape=jax.ShapeDtypeStruct(x.shape, x.dtype))(x)
    x = _gat(x, src, dst, p["W0"], p["al0"], p["ar0"], 8, 16)
    x = jnp.where(jnp.isinf(x), 1e9, x)
    x = _gat(x, src, dst, p["W1"], p["al1"], p["ar1"], 8, 16)
    s = jax.nn.softmax(x @ p["Wassign"], axis=-1)
    x = jnp.where(jnp.isinf(x), 1e9, x)
    x = _gat(x, src, dst, p["W2"], p["al2"], p["ar2"], 8, 16)
    x = jnp.where(jnp.isinf(x), 1e9, x)
    x = _gat(x, src, dst, p["W3"], p["al3"], p["ar3"], 1, 128)
    x = jnp.where(jnp.isinf(x), 1e9, x)
    he = jnp.concatenate([x[src], x[dst]], axis=1)
    y = jax.nn.relu(he @ p["fcW0"] + p["fcb0"])
    y = jax.nn.relu(y @ p["fcW1"] + p["fcb1"])
    y = y @ p["fcW2"] + p["fcb2"]
    y = jnp.where(jnp.isinf(y), 1e9, y)
    return (y, s)


# speedup vs baseline: 51.5732x; 47.1943x over previous
import functools

import numpy as np
import jax
import jax.numpy as jnp
from jax import lax
from jax.experimental import pallas as pl
from jax.experimental.pallas import tpu as pltpu
from jax.experimental.pallas import tpu_sc as plsc

_N = 10000
_E = 320000
_NBLK = 10
_BN = _N // _NBLK

_NC = 2
_NS = 16
_TILES = _NC * _NS
_EPT = _E // _TILES
_C = 80
_KB = 25
_NKB = _EPT // (_C * _KB)
_RPT = _EPT // _C
_NR = _N // _NS

_EBLK = 40
_BE = _E // _EBLK


def _embed_body(h_ref, w_ref, b_ref, o_ref):
    o_ref[...] = jnp.dot(h_ref[...], w_ref[...],
                         preferred_element_type=jnp.float32) + b_ref[...]


def _embed(h, W, b):
    return pl.pallas_call(
        _embed_body,
        grid=(_NBLK,),
        in_specs=[pl.BlockSpec((_BN, 128), lambda i: (i, 0)),
                  pl.BlockSpec((128, 128), lambda i: (0, 0)),
                  pl.BlockSpec((1, 128), lambda i: (0, 0))],
        out_specs=pl.BlockSpec((_BN, 128), lambda i: (i, 0)),
        out_shape=jax.ShapeDtypeStruct((_N, 128), jnp.float32),
    )(h, W, b.reshape(1, 128))


def _pre_body(x_ref, w_ref, al_ref, ar_ref, z_ref, el_ref, er_ref):
    x = x_ref[...]
    x = jnp.where(jnp.isinf(x), 1e9, x)
    z = jnp.dot(x, w_ref[...], preferred_element_type=jnp.float32)
    z_ref[...] = z
    el_ref[...] = jnp.dot(z, al_ref[...], preferred_element_type=jnp.float32)
    er_ref[...] = jnp.dot(z, ar_ref[...], preferred_element_type=jnp.float32)


def _pre(x, W, Aal, Aar):
    return pl.pallas_call(
        _pre_body,
        grid=(_NBLK,),
        in_specs=[pl.BlockSpec((_BN, 128), lambda i: (i, 0)),
                  pl.BlockSpec((128, 128), lambda i: (0, 0)),
                  pl.BlockSpec((128, 16), lambda i: (0, 0)),
                  pl.BlockSpec((128, 16), lambda i: (0, 0))],
        out_specs=[pl.BlockSpec((_BN, 128), lambda i: (i, 0)),
                   pl.BlockSpec((_BN, 16), lambda i: (i, 0)),
                   pl.BlockSpec((_BN, 16), lambda i: (i, 0))],
        out_shape=[jax.ShapeDtypeStruct((_N, 128), jnp.float32),
                   jax.ShapeDtypeStruct((_N, 16), jnp.float32),
                   jax.ShapeDtypeStruct((_N, 16), jnp.float32)],
    )(x, W, Aal, Aar)


def _fin_body(np_ref, dp_ref, x_ref, p_ref, o_ref):
    num = np_ref[0] + np_ref[1]
    den = dp_ref[0] + dp_ref[1]
    rec = 1.0 / (den + 1e-9)
    recx = jnp.dot(rec, p_ref[...], preferred_element_type=jnp.float32)
    o = num * recx
    o = jnp.where(o > 0, o, jnp.exp(jnp.minimum(o, 0.0)) - 1.0)
    x = x_ref[...]
    x = jnp.where(jnp.isinf(x), 1e9, x)
    o_ref[...] = o + x


def _fin(num, den, x, P):
    return pl.pallas_call(
        _fin_body,
        grid=(_NBLK,),
        in_specs=[pl.BlockSpec((_NC, _BN, 128), lambda i: (0, i, 0)),
                  pl.BlockSpec((_NC, _BN, 16), lambda i: (0, i, 0)),
                  pl.BlockSpec((_BN, 128), lambda i: (i, 0)),
                  pl.BlockSpec((16, 128), lambda i: (0, 0))],
        out_specs=pl.BlockSpec((_BN, 128), lambda i: (i, 0)),
        out_shape=jax.ShapeDtypeStruct((_N, 128), jnp.float32),
    )(num, den, x, P)


def _assign_body(x_ref, w_ref, o_ref):
    l = jnp.dot(x_ref[...], w_ref[...], preferred_element_type=jnp.float32)
    m = jnp.max(l, axis=1, keepdims=True)
    ex = jnp.exp(l - m)
    o_ref[...] = ex / jnp.sum(ex, axis=1, keepdims=True)


def _assign(x, Wassign):
    return pl.pallas_call(
        _assign_body,
        grid=(_NBLK,),
        in_specs=[pl.BlockSpec((_BN, 128), lambda i: (i, 0)),
                  pl.BlockSpec((128, 100), lambda i: (0, 0))],
        out_specs=pl.BlockSpec((_BN, 100), lambda i: (i, 0)),
        out_shape=jax.ShapeDtypeStruct((_N, 100), jnp.float32),
    )(x, Wassign)


def _ab_body(x_ref, wa_ref, wb_ref, b_ref, a_ref, bo_ref):
    x = x_ref[...]
    x = jnp.where(jnp.isinf(x), 1e9, x)
    a_ref[...] = jnp.dot(x, wa_ref[...],
                         preferred_element_type=jnp.float32) + b_ref[...]
    bo_ref[...] = jnp.dot(x, wb_ref[...], preferred_element_type=jnp.float32)


def _ab(x, Wa, Wb, b0):
    return pl.pallas_call(
        _ab_body,
        grid=(_NBLK,),
        in_specs=[pl.BlockSpec((_BN, 128), lambda i: (i, 0)),
                  pl.BlockSpec((128, 128), lambda i: (0, 0)),
                  pl.BlockSpec((128, 128), lambda i: (0, 0)),
                  pl.BlockSpec((1, 128), lambda i: (0, 0))],
        out_specs=[pl.BlockSpec((_BN, 128), lambda i: (i, 0)),
                   pl.BlockSpec((_BN, 128), lambda i: (i, 0))],
        out_shape=[jax.ShapeDtypeStruct((_N, 128), jnp.float32),
                   jax.ShapeDtypeStruct((_N, 128), jnp.float32)],
    )(x, Wa, Wb, b0.reshape(1, 128))


def _mlp_body(y1_ref, w1_ref, b1_ref, w2_ref, b2_ref, o_ref):
    t = jnp.dot(y1_ref[...], w1_ref[...],
                preferred_element_type=jnp.float32) + b1_ref[...]
    t = jnp.maximum(t, 0.0)
    y = jnp.dot(t, w2_ref[...], preferred_element_type=jnp.float32) + b2_ref[...]
    o_ref[...] = jnp.where(jnp.isinf(y), 1e9, y)


def _mlp(y1, W1, b1, W2, b2):
    return pl.pallas_call(
        _mlp_body,
        grid=(_EBLK,),
        in_specs=[pl.BlockSpec((_BE, 128), lambda i: (i, 0)),
                  pl.BlockSpec((128, 64), lambda i: (0, 0)),
                  pl.BlockSpec((1, 64), lambda i: (0, 0)),
                  pl.BlockSpec((64, 2), lambda i: (0, 0)),
                  pl.BlockSpec((1, 2), lambda i: (0, 0))],
        out_specs=pl.BlockSpec((_BE, 2), lambda i: (i, 0)),
        out_shape=jax.ShapeDtypeStruct((_E, 2), jnp.float32),
    )(y1, W1, b1.reshape(1, 64), W2, b2.reshape(1, 2))


def _make_edge_pass(H):
    cols = [(g * H) // 8 for g in range(8)]
    mesh = plsc.VectorSubcoreMesh(core_axis_name="c", subcore_axis_name="s", num_cores=_NC, num_subcores=_NS)

    def body(src_hbm, dst_hbm, z_hbm, el_hbm, er_hbm, z128_hbm, z16_hbm,
             num_out, den_out,
             snum, sden, srcv, dstv, elr, err, exv, zr, sem0, sem1, sem2):
        c = lax.axis_index("c")
        s = lax.axis_index("s")
        wid = s * _NC + c
        @pl.when(s < 15)
        def _():
            pltpu.sync_copy(z128_hbm.at[pl.ds(s * 624, 624)],
                            snum.at[pl.ds(s * 624, 624)])
            pltpu.sync_copy(z16_hbm.at[pl.ds(s * 624, 624)],
                            sden.at[pl.ds(s * 624, 624)])

        @pl.when(s == 15)
        def _():
            pltpu.sync_copy(z128_hbm.at[pl.ds(9360, 640)],
                            snum.at[pl.ds(9360, 640)])
            pltpu.sync_copy(z16_hbm.at[pl.ds(9360, 640)],
                            sden.at[pl.ds(9360, 640)])

        plsc.subcore_barrier()

        row0 = wid * _RPT

        def kb_body(kb, carry):
            pltpu.sync_copy(src_hbm.at[wid * _NKB + kb], srcv)
            pltpu.sync_copy(dst_hbm.at[wid * _NKB + kb], dstv)

            def j_body(j, carry2):
                si = srcv.at[j]
                di = dstv.at[j]
                c1 = pltpu.async_copy(el_hbm.at[si], elr, sem0)
                c2 = pltpu.async_copy(er_hbm.at[di], err, sem1)
                c3 = pltpu.async_copy(z_hbm.at[si], zr, sem2)
                c1.wait()
                c2.wait()
                c3.wait()

                def e_body(i, carry3):
                    lg = elr[i] + err[i]
                    lg = jnp.where(lg >= 0, lg, 0.2 * lg)
                    ex = jnp.exp(lg)
                    exv[i] = ex
                    for g in range(8):
                        mult = ex[jnp.full((16,), cols[g], jnp.int32)]
                        sl = pl.ds(g * 16, 16)
                        zr[i, sl] = zr[i, sl] * mult
                    return 0

                lax.fori_loop(0, _C, e_body, 0)
                pltpu.sync_copy(zr, snum.at[di], add=True)
                pltpu.sync_copy(exv, sden.at[di], add=True)
                return 0

            lax.fori_loop(0, _KB, j_body, 0)
            return 0

        lax.fori_loop(0, _NKB, kb_body, 0)

        plsc.subcore_barrier()

        @pl.when(s < 15)
        def _():
            pltpu.sync_copy(snum.at[pl.ds(s * 624, 624)],
                            num_out.at[c, pl.ds(s * 624, 624)])
            pltpu.sync_copy(sden.at[pl.ds(s * 624, 624)],
                            den_out.at[c, pl.ds(s * 624, 624)])

        @pl.when(s == 15)
        def _():
            pltpu.sync_copy(snum.at[pl.ds(9360, 640)],
                            num_out.at[c, pl.ds(9360, 640)])
            pltpu.sync_copy(sden.at[pl.ds(9360, 640)],
                            den_out.at[c, pl.ds(9360, 640)])

    return pl.kernel(
        body,
        out_type=(jax.ShapeDtypeStruct((_NC, _N, 128), jnp.float32),
                  jax.ShapeDtypeStruct((_NC, _N, 16), jnp.float32)),
        mesh=mesh,
        compiler_params=pltpu.CompilerParams(use_tc_tiling_on_sc=False),
        scratch_types=[
            pltpu.VMEM_SHARED((_N, 128), jnp.float32),
            pltpu.VMEM_SHARED((_N, 16), jnp.float32),
            pltpu.VMEM((_KB, _C), jnp.int32),
            pltpu.VMEM((_KB, _C), jnp.int32),
            pltpu.VMEM((_C, 16), jnp.float32),
            pltpu.VMEM((_C, 16), jnp.float32),
            pltpu.VMEM((_C, 16), jnp.float32),
            pltpu.VMEM((_C, 128), jnp.float32),
            pltpu.SemaphoreType.DMA,
            pltpu.SemaphoreType.DMA,
            pltpu.SemaphoreType.DMA,
        ],
    )


@functools.lru_cache(maxsize=None)
def _edge_pass(H):
    return _make_edge_pass(H)


def _edge_cat_body(src_hbm, dst_hbm, a_hbm, b_hbm, y1_out,
                   srcv, dstv, ar, br, sem0, sem1):
    c = lax.axis_index("c")
    s = lax.axis_index("s")
    wid = s * _NC + c
    row0 = wid * _RPT

    def kb_body(kb, carry):
        pltpu.sync_copy(src_hbm.at[wid * _NKB + kb], srcv)
        pltpu.sync_copy(dst_hbm.at[wid * _NKB + kb], dstv)

        def j_body(j, carry2):
            si = srcv.at[j]
            di = dstv.at[j]
            c1 = pltpu.async_copy(a_hbm.at[si], ar, sem0)
            c2 = pltpu.async_copy(b_hbm.at[di], br, sem1)
            c1.wait()
            c2.wait()

            def e_body(i, carry3):
                for g in range(8):
                    sl = pl.ds(g * 16, 16)
                    ar[i, sl] = jnp.maximum(ar[i, sl] + br[i, sl], 0.0)
                return 0

            lax.fori_loop(0, _C, e_body, 0)
            erow = (row0 + kb * _KB + j) * _C
            pltpu.sync_copy(ar, y1_out.at[pl.ds(erow, _C)])
            return 0

        lax.fori_loop(0, _KB, j_body, 0)
        return 0

    lax.fori_loop(0, _NKB, kb_body, 0)


@functools.lru_cache(maxsize=None)
def _edge_cat_kernel():
    return pl.kernel(
        _edge_cat_body,
        out_type=jax.ShapeDtypeStruct((_E, 128), jnp.float32),
        mesh=plsc.VectorSubcoreMesh(core_axis_name="c", subcore_axis_name="s",
                                    num_cores=_NC, num_subcores=_NS),
        scratch_types=[
            pltpu.VMEM((_KB, _C), jnp.int32),
            pltpu.VMEM((_KB, _C), jnp.int32),
            pltpu.VMEM((_C, 128), jnp.float32),
            pltpu.VMEM((_C, 128), jnp.float32),
            pltpu.SemaphoreType.DMA,
            pltpu.SemaphoreType.DMA,
        ],
    )


def _dup_table_mat(al, H):
    d = 128 // H
    k = np.arange(128)
    mask = np.equal(k[:, None] // d, np.arange(H)[None, :]).astype(np.float32)
    blkdiag = al.T[k % d, :] * mask
    return blkdiag[:, np.arange(16) % H]


def _expand_mat(H):
    d = 128 // H
    P = np.zeros((16, 128), np.float32)
    for f in range(128):
        P[f // d, f] = 1.0
    return jnp.asarray(P)


def kernel(h, e, edge_index, params):
    p = params
    src2 = edge_index[0].reshape(_TILES * _NKB, _KB, _C)
    dst2 = edge_index[1].reshape(_TILES * _NKB, _KB, _C)
    z128 = jnp.zeros((_N, 128), jnp.float32)
    z16 = jnp.zeros((_N, 16), jnp.float32)

    x = _embed(h, p["W_emb"], p["b_emb"])

    layers = [(p["W0"], p["al0"], p["ar0"], 8),
              (p["W1"], p["al1"], p["ar1"], 8),
              (p["W2"], p["al2"], p["ar2"], 8),
              (p["W3"], p["al3"], p["ar3"], 1)]

    s_out = None
    for li, (W, al, ar, H) in enumerate(layers):
        Aal = _dup_table_mat(al, H)
        Aar = _dup_table_mat(ar, H)
        z, eld, erd = _pre(x, W, Aal, Aar)
        num, den = _edge_pass(H)(src2, dst2, z, eld, erd, z128, z16)
        x = _fin(num, den, x, _expand_mat(H))
        if li == 1:
            s_out = _assign(x, p["Wassign"])

    A, B = _ab(x, p["fcW0"][:128], p["fcW0"][128:], p["fcb0"])
    y1 = _edge_cat_kernel()(src2, dst2, A, B)
    y = _mlp(y1, p["fcW1"], p["fcb1"], p["fcW2"], p["fcb2"])
    return (y, s_out)


# speedup vs baseline: 69.3092x; 1.3439x over previous
import functools

import numpy as np
import jax
import jax.numpy as jnp
from jax import lax
from jax.experimental import pallas as pl
from jax.experimental.pallas import tpu as pltpu
from jax.experimental.pallas import tpu_sc as plsc

_N = 10000
_E = 320000
_NBLK = 10
_BN = _N // _NBLK

_NC = 2
_NS = 16
_TILES = _NC * _NS
_EPT = _E // _TILES
_C = 80
_KB = 25
_NKB = _EPT // (_C * _KB)
_RPT = _EPT // _C
_NR = _N // _NS

_EBLK = 40
_BE = _E // _EBLK


def _embed_body(h_ref, w_ref, b_ref, o_ref):
    o_ref[...] = jnp.dot(h_ref[...], w_ref[...],
                         preferred_element_type=jnp.float32) + b_ref[...]


def _embed(h, W, b):
    return pl.pallas_call(
        _embed_body,
        grid=(_NBLK,),
        in_specs=[pl.BlockSpec((_BN, 128), lambda i: (i, 0)),
                  pl.BlockSpec((128, 128), lambda i: (0, 0)),
                  pl.BlockSpec((1, 128), lambda i: (0, 0))],
        out_specs=pl.BlockSpec((_BN, 128), lambda i: (i, 0)),
        out_shape=jax.ShapeDtypeStruct((_N, 128), jnp.float32),
    )(h, W, b.reshape(1, 128))


def _pre_body(x_ref, w_ref, al_ref, ar_ref, zel_ref, er_ref):
    x = x_ref[...]
    x = jnp.where(jnp.isinf(x), 1e9, x)
    z = jnp.dot(x, w_ref[...], preferred_element_type=jnp.float32)
    el = jnp.dot(z, al_ref[...], preferred_element_type=jnp.float32)
    zel_ref[...] = jnp.concatenate([z, el], axis=1)
    er_ref[...] = jnp.dot(z, ar_ref[...], preferred_element_type=jnp.float32)


def _pre(x, W, Aal, Aar):
    return pl.pallas_call(
        _pre_body,
        grid=(_NBLK,),
        in_specs=[pl.BlockSpec((_BN, 128), lambda i: (i, 0)),
                  pl.BlockSpec((128, 128), lambda i: (0, 0)),
                  pl.BlockSpec((128, 16), lambda i: (0, 0)),
                  pl.BlockSpec((128, 16), lambda i: (0, 0))],
        out_specs=[pl.BlockSpec((_BN, 144), lambda i: (i, 0)),
                   pl.BlockSpec((_BN, 16), lambda i: (i, 0))],
        out_shape=[jax.ShapeDtypeStruct((_N, 144), jnp.float32),
                   jax.ShapeDtypeStruct((_N, 16), jnp.float32)],
    )(x, W, Aal, Aar)


def _fin_body(acc_ref, x_ref, p_ref, o_ref):
    t = acc_ref[0] + acc_ref[1]
    num = t[:, :128]
    den = t[:, 128:]
    rec = 1.0 / (den + 1e-9)
    recx = jnp.dot(rec, p_ref[...], preferred_element_type=jnp.float32)
    o = num * recx
    o = jnp.where(o > 0, o, jnp.exp(jnp.minimum(o, 0.0)) - 1.0)
    x = x_ref[...]
    x = jnp.where(jnp.isinf(x), 1e9, x)
    o_ref[...] = o + x


def _fin(acc, x, P):
    return pl.pallas_call(
        _fin_body,
        grid=(_NBLK,),
        in_specs=[pl.BlockSpec((_NC, _BN, 144), lambda i: (0, i, 0)),
                  pl.BlockSpec((_BN, 128), lambda i: (i, 0)),
                  pl.BlockSpec((16, 128), lambda i: (0, 0))],
        out_specs=pl.BlockSpec((_BN, 128), lambda i: (i, 0)),
        out_shape=jax.ShapeDtypeStruct((_N, 128), jnp.float32),
    )(acc, x, P)


def _assign_body(x_ref, w_ref, o_ref):
    l = jnp.dot(x_ref[...], w_ref[...], preferred_element_type=jnp.float32)
    m = jnp.max(l, axis=1, keepdims=True)
    ex = jnp.exp(l - m)
    o_ref[...] = ex / jnp.sum(ex, axis=1, keepdims=True)


def _assign(x, Wassign):
    return pl.pallas_call(
        _assign_body,
        grid=(_NBLK,),
        in_specs=[pl.BlockSpec((_BN, 128), lambda i: (i, 0)),
                  pl.BlockSpec((128, 100), lambda i: (0, 0))],
        out_specs=pl.BlockSpec((_BN, 100), lambda i: (i, 0)),
        out_shape=jax.ShapeDtypeStruct((_N, 100), jnp.float32),
    )(x, Wassign)


def _ab_body(x_ref, wa_ref, wb_ref, b_ref, a_ref, bo_ref):
    x = x_ref[...]
    x = jnp.where(jnp.isinf(x), 1e9, x)
    a_ref[...] = jnp.dot(x, wa_ref[...],
                         preferred_element_type=jnp.float32) + b_ref[...]
    bo_ref[...] = jnp.dot(x, wb_ref[...], preferred_element_type=jnp.float32)


def _ab(x, Wa, Wb, b0):
    return pl.pallas_call(
        _ab_body,
        grid=(_NBLK,),
        in_specs=[pl.BlockSpec((_BN, 128), lambda i: (i, 0)),
                  pl.BlockSpec((128, 128), lambda i: (0, 0)),
                  pl.BlockSpec((128, 128), lambda i: (0, 0)),
                  pl.BlockSpec((1, 128), lambda i: (0, 0))],
        out_specs=[pl.BlockSpec((_BN, 128), lambda i: (i, 0)),
                   pl.BlockSpec((_BN, 128), lambda i: (i, 0))],
        out_shape=[jax.ShapeDtypeStruct((_N, 128), jnp.float32),
                   jax.ShapeDtypeStruct((_N, 128), jnp.float32)],
    )(x, Wa, Wb, b0.reshape(1, 128))


def _mlp_body(y1_ref, w1_ref, b1_ref, w2_ref, b2_ref, o_ref):
    y1 = jnp.maximum(y1_ref[...], 0.0)
    t = jnp.dot(y1, w1_ref[...],
                preferred_element_type=jnp.float32) + b1_ref[...]
    t = jnp.maximum(t, 0.0)
    y = jnp.dot(t, w2_ref[...], preferred_element_type=jnp.float32) + b2_ref[...]
    o_ref[...] = jnp.where(jnp.isinf(y), 1e9, y)


def _mlp(y1, W1, b1, W2, b2):
    return pl.pallas_call(
        _mlp_body,
        grid=(_EBLK,),
        in_specs=[pl.BlockSpec((_BE, 128), lambda i: (i, 0)),
                  pl.BlockSpec((128, 64), lambda i: (0, 0)),
                  pl.BlockSpec((1, 64), lambda i: (0, 0)),
                  pl.BlockSpec((64, 2), lambda i: (0, 0)),
                  pl.BlockSpec((1, 2), lambda i: (0, 0))],
        out_specs=pl.BlockSpec((_BE, 2), lambda i: (i, 0)),
        out_shape=jax.ShapeDtypeStruct((_E, 2), jnp.float32),
    )(y1, W1, b1.reshape(1, 64), W2, b2.reshape(1, 2))


_RING = 5
_CP = 20
_NCHP = _EPT // _CP
_OUTERP = _NCHP // _RING
_NCH = _EPT // _C
_OUTER = _NCH // _RING


def _make_edge_pass(H):
    cols = [(g * H) // 8 for g in range(8)]
    mesh = plsc.VectorSubcoreMesh(core_axis_name="c", subcore_axis_name="s",
                                  num_cores=_NC, num_subcores=_NS)

    def body(src_hbm, dst_hbm, zel_hbm, er_hbm, zero_hbm, acc_out,
             sacc, srcv, dstv,
             zsr0, zsr1, zsr2, zsr3, zsr4,
             err0, err1, err2, err3, err4,
             sg0, sg1, sg2, sg3, sg4,
             se0, se1, se2, se3, se4,
             ss0, ss1, ss2, ss3, ss4):
        zsr = [zsr0, zsr1, zsr2, zsr3, zsr4]
        errb = [err0, err1, err2, err3, err4]
        sg = [sg0, sg1, sg2, sg3, sg4]
        se = [se0, se1, se2, se3, se4]
        ss = [ss0, ss1, ss2, ss3, ss4]

        c = lax.axis_index("c")
        s = lax.axis_index("s")
        wid = s * _NC + c

        @pl.when(s < 15)
        def _():
            pltpu.sync_copy(zero_hbm.at[pl.ds(s * 624, 624)],
                            sacc.at[pl.ds(s * 624, 624)])

        @pl.when(s == 15)
        def _():
            pltpu.sync_copy(zero_hbm.at[pl.ds(9360, 640)],
                            sacc.at[pl.ds(9360, 640)])

        pltpu.sync_copy(src_hbm.at[wid], srcv)
        pltpu.sync_copy(dst_hbm.at[wid], dstv)
        plsc.subcore_barrier()

        def issue_g(j, b):
            pltpu.async_copy(zel_hbm.at[srcv.at[j]], zsr[b], sg[b])
            pltpu.async_copy(er_hbm.at[dstv.at[j]], errb[b], se[b])

        def wait_g(j, b):
            pltpu.make_async_copy(zel_hbm.at[srcv.at[j]], zsr[b], sg[b]).wait()
            pltpu.make_async_copy(er_hbm.at[dstv.at[j]], errb[b], se[b]).wait()

        def issue_s(j, b):
            pltpu.async_copy(zsr[b], sacc.at[dstv.at[j]], ss[b], add=True)

        def wait_s(j, b):
            pltpu.make_async_copy(zsr[b], sacc.at[dstv.at[j]], ss[b]).wait()

        issue_g(0, 0)
        issue_g(1, 1)

        def outer(jo, carry):
            for b in range(_RING):
                j = jo * _RING + b
                wait_g(j, b)
                zb = zsr[b]
                eb = errb[b]

                def e_body(i, carry3, zb=zb, eb=eb):
                    lg = zb[i, pl.ds(128, 16)] + eb[i]
                    lg = jnp.maximum(lg, 0.2 * lg)
                    ex = jnp.exp(lg)
                    zb[i, pl.ds(128, 16)] = ex
                    for g in range(8):
                        mult = ex[jnp.full((16,), cols[g], jnp.int32)]
                        sl = pl.ds(g * 16, 16)
                        zb[i, sl] = zb[i, sl] * mult
                    return 0

                lax.fori_loop(0, _CP, e_body, 0)
                issue_s(j, b)

                @pl.when(j >= 2)
                def _(b=b, j=j):
                    wait_s(j - 2, (b - 2) % _RING)

                @pl.when(j <= _NCHP - 3)
                def _(b=b, j=j):
                    issue_g(j + 2, (b + 2) % _RING)
            return 0

        lax.fori_loop(0, _OUTERP, outer, 0)
        wait_s(_NCHP - 2, (_NCHP - 2) % _RING)
        wait_s(_NCHP - 1, (_NCHP - 1) % _RING)

        plsc.subcore_barrier()

        @pl.when(s < 15)
        def _():
            pltpu.sync_copy(sacc.at[pl.ds(s * 624, 624)],
                            acc_out.at[c, pl.ds(s * 624, 624)])

        @pl.when(s == 15)
        def _():
            pltpu.sync_copy(sacc.at[pl.ds(9360, 640)],
                            acc_out.at[c, pl.ds(9360, 640)])

    return pl.kernel(
        body,
        out_type=jax.ShapeDtypeStruct((_NC, _N, 144), jnp.float32),
        mesh=mesh,
        compiler_params=pltpu.CompilerParams(use_tc_tiling_on_sc=False),
        scratch_types=(
            [pltpu.VMEM_SHARED((_N, 144), jnp.float32),
             pltpu.VMEM((_NCHP, _CP), jnp.int32),
             pltpu.VMEM((_NCHP, _CP), jnp.int32)]
            + [pltpu.VMEM((_CP, 144), jnp.float32) for _ in range(_RING)]
            + [pltpu.VMEM((_CP, 16), jnp.float32) for _ in range(_RING)]
            + [pltpu.SemaphoreType.DMA for _ in range(3 * _RING)]
        ),
    )


@functools.lru_cache(maxsize=None)
def _edge_pass(H):
    return _make_edge_pass(H)


def _edge_cat_body(src_hbm, dst_hbm, a_hbm, b_hbm, y1_out,
                   srcv, dstv,
                   ba0, ba1, ba2, ba3, ba4,
                   sa0, sa1, sa2, sa3, sa4,
                   sb0, sb1, sb2, sb3, sb4,
                   sw0, sw1, sw2, sw3, sw4):
    ba = [ba0, ba1, ba2, ba3, ba4]
    sa = [sa0, sa1, sa2, sa3, sa4]
    sb = [sb0, sb1, sb2, sb3, sb4]
    sw = [sw0, sw1, sw2, sw3, sw4]

    c = lax.axis_index("c")
    s = lax.axis_index("s")
    wid = s * _NC + c

    pltpu.sync_copy(src_hbm.at[wid], srcv)
    pltpu.sync_copy(dst_hbm.at[wid], dstv)

    def issue_a(j, b):
        pltpu.async_copy(a_hbm.at[srcv.at[j]], ba[b], sa[b])

    def wait_a(j, b):
        pltpu.make_async_copy(a_hbm.at[srcv.at[j]], ba[b], sa[b]).wait()

    def issue_b(j, b):
        pltpu.async_copy(b_hbm.at[dstv.at[j]], ba[b], sb[b], add=True)

    def wait_b(j, b):
        pltpu.make_async_copy(b_hbm.at[dstv.at[j]], ba[b], sb[b]).wait()

    def issue_w(j, b):
        pltpu.async_copy(ba[b], y1_out.at[pl.ds((wid * _NCH + j) * _C, _C)],
                         sw[b])

    def wait_w(j, b):
        pltpu.make_async_copy(ba[b],
                              y1_out.at[pl.ds((wid * _NCH + j) * _C, _C)],
                              sw[b]).wait()

    issue_a(0, 0)
    issue_a(1, 1)
    wait_a(0, 0)
    issue_b(0, 0)

    def outer(jo, carry):
        for b in range(_RING):
            j = jo * _RING + b

            @pl.when(j >= 2)
            def _(b=b, j=j):
                wait_w(j - 2, (b - 2) % _RING)

            @pl.when(j <= _NCH - 3)
            def _(b=b, j=j):
                issue_a(j + 2, (b + 2) % _RING)

            @pl.when(j <= _NCH - 2)
            def _(b=b, j=j):
                wait_a(j + 1, (b + 1) % _RING)
                issue_b(j + 1, (b + 1) % _RING)

            wait_b(j, b)
            issue_w(j, b)
        return 0

    lax.fori_loop(0, _OUTER, outer, 0)
    wait_w(_NCH - 2, (_NCH - 2) % _RING)
    wait_w(_NCH - 1, (_NCH - 1) % _RING)


@functools.lru_cache(maxsize=None)
def _edge_cat_kernel():
    return pl.kernel(
        _edge_cat_body,
        out_type=jax.ShapeDtypeStruct((_E, 128), jnp.float32),
        mesh=plsc.VectorSubcoreMesh(core_axis_name="c", subcore_axis_name="s",
                                    num_cores=_NC, num_subcores=_NS),
        scratch_types=(
            [pltpu.VMEM((_NCH, _C), jnp.int32),
             pltpu.VMEM((_NCH, _C), jnp.int32)]
            + [pltpu.VMEM((_C, 128), jnp.float32) for _ in range(_RING)]
            + [pltpu.SemaphoreType.DMA for _ in range(3 * _RING)]
        ),
    )


def _dup_table_mat(al, H):
    d = 128 // H
    k = np.arange(128)
    mask = np.equal(k[:, None] // d, np.arange(H)[None, :]).astype(np.float32)
    blkdiag = al.T[k % d, :] * mask
    return blkdiag[:, np.arange(16) % H]


def _expand_mat(H):
    d = 128 // H
    P = np.zeros((16, 128), np.float32)
    for f in range(128):
        P[f // d, f] = 1.0
    return jnp.asarray(P)


def kernel(h, e, edge_index, params):
    p = params
    srcp = edge_index[0].reshape(_TILES, _NCHP, _CP)
    dstp = edge_index[1].reshape(_TILES, _NCHP, _CP)
    src2 = edge_index[0].reshape(_TILES, _NCH, _C)
    dst2 = edge_index[1].reshape(_TILES, _NCH, _C)
    z144 = jnp.zeros((_N, 144), jnp.float32)

    x = _embed(h, p["W_emb"], p["b_emb"])

    layers = [(p["W0"], p["al0"], p["ar0"], 8),
              (p["W1"], p["al1"], p["ar1"], 8),
              (p["W2"], p["al2"], p["ar2"], 8),
              (p["W3"], p["al3"], p["ar3"], 1)]

    s_out = None
    for li, (W, al, ar, H) in enumerate(layers):
        Aal = _dup_table_mat(al, H)
        Aar = _dup_table_mat(ar, H)
        zel, erd = _pre(x, W, Aal, Aar)
        acc = _edge_pass(H)(srcp, dstp, zel, erd, z144)
        x = _fin(acc, x, _expand_mat(H))
        if li == 1:
            s_out = _assign(x, p["Wassign"])

    A, B = _ab(x, p["fcW0"][:128], p["fcW0"][128:], p["fcb0"])
    y1 = _edge_cat_kernel()(src2, dst2, A, B)
    y = _mlp(y1, p["fcW1"], p["fcb1"], p["fcW2"], p["fcb2"])
    return (y, s_out)


# speedup vs baseline: 79.3520x; 1.1449x over previous
import functools

import numpy as np
import jax
import jax.numpy as jnp
from jax import lax
from jax.experimental import pallas as pl
from jax.experimental.pallas import tpu as pltpu
from jax.experimental.pallas import tpu_sc as plsc

_N = 10000
_E = 320000
_NBLK = 10
_BN = _N // _NBLK

_NC = 2
_NS = 16
_TILES = _NC * _NS
_EPT = _E // _TILES
_C = 80
_KB = 25
_NKB = _EPT // (_C * _KB)
_RPT = _EPT // _C
_NR = _N // _NS

_EBLK = 40
_BE = _E // _EBLK


def _embed_body(h_ref, w_ref, b_ref, o_ref):
    o_ref[...] = jnp.dot(h_ref[...], w_ref[...],
                         preferred_element_type=jnp.float32) + b_ref[...]


def _embed(h, W, b):
    return pl.pallas_call(
        _embed_body,
        grid=(_NBLK,),
        in_specs=[pl.BlockSpec((_BN, 128), lambda i: (i, 0)),
                  pl.BlockSpec((128, 128), lambda i: (0, 0)),
                  pl.BlockSpec((1, 128), lambda i: (0, 0))],
        out_specs=pl.BlockSpec((_BN, 128), lambda i: (i, 0)),
        out_shape=jax.ShapeDtypeStruct((_N, 128), jnp.float32),
    )(h, W, b.reshape(1, 128))


def _pre_body(x_ref, w_ref, al_ref, ar_ref, zel_ref, er_ref):
    x = x_ref[...]
    x = jnp.where(jnp.isinf(x), 1e9, x)
    z = jnp.dot(x, w_ref[...], preferred_element_type=jnp.float32)
    el = jnp.dot(z, al_ref[...], preferred_element_type=jnp.float32)
    zel_ref[...] = jnp.concatenate([z, el], axis=1)
    er_ref[...] = jnp.dot(z, ar_ref[...], preferred_element_type=jnp.float32)


def _pre(x, W, Aal, Aar):
    return pl.pallas_call(
        _pre_body,
        grid=(_NBLK,),
        in_specs=[pl.BlockSpec((_BN, 128), lambda i: (i, 0)),
                  pl.BlockSpec((128, 128), lambda i: (0, 0)),
                  pl.BlockSpec((128, 16), lambda i: (0, 0)),
                  pl.BlockSpec((128, 16), lambda i: (0, 0))],
        out_specs=[pl.BlockSpec((_BN, 144), lambda i: (i, 0)),
                   pl.BlockSpec((_BN, 16), lambda i: (i, 0))],
        out_shape=[jax.ShapeDtypeStruct((_N, 144), jnp.float32),
                   jax.ShapeDtypeStruct((_N, 16), jnp.float32)],
    )(x, W, Aal, Aar)


def _fin_body(acc_ref, x_ref, p_ref, o_ref):
    t = acc_ref[0] + acc_ref[1]
    num = t[:, :128]
    den = t[:, 128:]
    rec = 1.0 / (den + 1e-9)
    recx = jnp.dot(rec, p_ref[...], preferred_element_type=jnp.float32)
    o = num * recx
    o = jnp.where(o > 0, o, jnp.exp(jnp.minimum(o, 0.0)) - 1.0)
    x = x_ref[...]
    x = jnp.where(jnp.isinf(x), 1e9, x)
    o_ref[...] = o + x


def _fin(acc, x, P):
    return pl.pallas_call(
        _fin_body,
        grid=(_NBLK,),
        in_specs=[pl.BlockSpec((_NC, _BN, 144), lambda i: (0, i, 0)),
                  pl.BlockSpec((_BN, 128), lambda i: (i, 0)),
                  pl.BlockSpec((16, 128), lambda i: (0, 0))],
        out_specs=pl.BlockSpec((_BN, 128), lambda i: (i, 0)),
        out_shape=jax.ShapeDtypeStruct((_N, 128), jnp.float32),
    )(acc, x, P)


def _assign_body(x_ref, w_ref, o_ref):
    l = jnp.dot(x_ref[...], w_ref[...], preferred_element_type=jnp.float32)
    m = jnp.max(l, axis=1, keepdims=True)
    ex = jnp.exp(l - m)
    o_ref[...] = ex / jnp.sum(ex, axis=1, keepdims=True)


def _assign(x, Wassign):
    return pl.pallas_call(
        _assign_body,
        grid=(_NBLK,),
        in_specs=[pl.BlockSpec((_BN, 128), lambda i: (i, 0)),
                  pl.BlockSpec((128, 100), lambda i: (0, 0))],
        out_specs=pl.BlockSpec((_BN, 100), lambda i: (i, 0)),
        out_shape=jax.ShapeDtypeStruct((_N, 100), jnp.float32),
    )(x, Wassign)


def _ab_body(x_ref, wa_ref, wb_ref, b_ref, a_ref, bo_ref):
    x = x_ref[...]
    x = jnp.where(jnp.isinf(x), 1e9, x)
    a_ref[...] = jnp.dot(x, wa_ref[...],
                         preferred_element_type=jnp.float32) + b_ref[...]
    bo_ref[...] = jnp.dot(x, wb_ref[...], preferred_element_type=jnp.float32)


def _ab(x, Wa, Wb, b0):
    return pl.pallas_call(
        _ab_body,
        grid=(_NBLK,),
        in_specs=[pl.BlockSpec((_BN, 128), lambda i: (i, 0)),
                  pl.BlockSpec((128, 128), lambda i: (0, 0)),
                  pl.BlockSpec((128, 128), lambda i: (0, 0)),
                  pl.BlockSpec((1, 128), lambda i: (0, 0))],
        out_specs=[pl.BlockSpec((_BN, 128), lambda i: (i, 0)),
                   pl.BlockSpec((_BN, 128), lambda i: (i, 0))],
        out_shape=[jax.ShapeDtypeStruct((_N, 128), jnp.float32),
                   jax.ShapeDtypeStruct((_N, 128), jnp.float32)],
    )(x, Wa, Wb, b0.reshape(1, 128))


def _mlp_body(y1_ref, w1_ref, b1_ref, w2_ref, b2_ref, o_ref):
    y1 = jnp.maximum(y1_ref[...], 0.0)
    t = jnp.dot(y1, w1_ref[...],
                preferred_element_type=jnp.float32) + b1_ref[...]
    t = jnp.maximum(t, 0.0)
    y = jnp.dot(t, w2_ref[...], preferred_element_type=jnp.float32) + b2_ref[...]
    o_ref[...] = jnp.where(jnp.isinf(y), 1e9, y)


def _mlp(y1, W1, b1, W2, b2):
    return pl.pallas_call(
        _mlp_body,
        grid=(_EBLK,),
        in_specs=[pl.BlockSpec((_BE, 128), lambda i: (i, 0)),
                  pl.BlockSpec((128, 64), lambda i: (0, 0)),
                  pl.BlockSpec((1, 64), lambda i: (0, 0)),
                  pl.BlockSpec((64, 2), lambda i: (0, 0)),
                  pl.BlockSpec((1, 2), lambda i: (0, 0))],
        out_specs=pl.BlockSpec((_BE, 2), lambda i: (i, 0)),
        out_shape=jax.ShapeDtypeStruct((_E, 2), jnp.float32),
    )(y1, W1, b1.reshape(1, 64), W2, b2.reshape(1, 2))


_RING = 5
_CP = 20
_NCHP = _EPT // _CP
_OUTERP = _NCHP // _RING
_NCH = _EPT // _C
_OUTER = _NCH // _RING


def _make_edge_pass(H):
    cols = [(g * H) // 8 for g in range(8)]
    mesh = plsc.VectorSubcoreMesh(core_axis_name="c", subcore_axis_name="s",
                                  num_cores=_NC, num_subcores=_NS)

    def body(src_hbm, dst_hbm, zel_hbm, er_hbm, zero_hbm, acc_out,
             sacc, srcv, dstv,
             zsr0, zsr1, zsr2, zsr3, zsr4,
             err0, err1, err2, err3, err4,
             sg0, sg1, sg2, sg3, sg4,
             se0, se1, se2, se3, se4,
             ss0, ss1, ss2, ss3, ss4):
        zsr = [zsr0, zsr1, zsr2, zsr3, zsr4]
        errb = [err0, err1, err2, err3, err4]
        sg = [sg0, sg1, sg2, sg3, sg4]
        se = [se0, se1, se2, se3, se4]
        ss = [ss0, ss1, ss2, ss3, ss4]

        c = lax.axis_index("c")
        s = lax.axis_index("s")
        wid = s * _NC + c

        @pl.when(s < 15)
        def _():
            pltpu.sync_copy(zero_hbm.at[pl.ds(s * 624, 624)],
                            sacc.at[pl.ds(s * 624, 624)])

        @pl.when(s == 15)
        def _():
            pltpu.sync_copy(zero_hbm.at[pl.ds(9360, 640)],
                            sacc.at[pl.ds(9360, 640)])

        pltpu.sync_copy(src_hbm.at[wid], srcv)
        pltpu.sync_copy(dst_hbm.at[wid], dstv)
        plsc.subcore_barrier()

        def issue_g(j, b):
            pltpu.async_copy(zel_hbm.at[srcv.at[j]], zsr[b], sg[b])
            pltpu.async_copy(er_hbm.at[dstv.at[j]], errb[b], se[b])

        def wait_g(j, b):
            pltpu.make_async_copy(zel_hbm.at[srcv.at[j]], zsr[b], sg[b]).wait()
            pltpu.make_async_copy(er_hbm.at[dstv.at[j]], errb[b], se[b]).wait()

        def issue_s(j, b):
            pltpu.async_copy(zsr[b], sacc.at[dstv.at[j]], ss[b], add=True)

        def wait_s(j, b):
            pltpu.make_async_copy(zsr[b], sacc.at[dstv.at[j]], ss[b]).wait()

        issue_g(0, 0)
        issue_g(1, 1)

        def outer(jo, carry):
            for b in range(_RING):
                j = jo * _RING + b
                wait_g(j, b)
                zb = zsr[b]
                eb = errb[b]

                @plsc.parallel_loop(0, _CP, 1, unroll=4)
                def _(i, zb=zb, eb=eb):
                    lg = zb[i, pl.ds(128, 16)] + eb[i]
                    lg = jnp.maximum(lg, 0.2 * lg)
                    ex = jnp.exp(lg)
                    zb[i, pl.ds(128, 16)] = ex
                    for g in range(8):
                        mult = ex[jnp.full((16,), cols[g], jnp.int32)]
                        sl = pl.ds(g * 16, 16)
                        zb[i, sl] = zb[i, sl] * mult
                issue_s(j, b)

                @pl.when(j >= 2)
                def _(b=b, j=j):
                    wait_s(j - 2, (b - 2) % _RING)

                @pl.when(j <= _NCHP - 3)
                def _(b=b, j=j):
                    issue_g(j + 2, (b + 2) % _RING)
            return 0

        lax.fori_loop(0, _OUTERP, outer, 0)
        wait_s(_NCHP - 2, (_NCHP - 2) % _RING)
        wait_s(_NCHP - 1, (_NCHP - 1) % _RING)

        plsc.subcore_barrier()

        @pl.when(s < 15)
        def _():
            pltpu.sync_copy(sacc.at[pl.ds(s * 624, 624)],
                            acc_out.at[c, pl.ds(s * 624, 624)])

        @pl.when(s == 15)
        def _():
            pltpu.sync_copy(sacc.at[pl.ds(9360, 640)],
                            acc_out.at[c, pl.ds(9360, 640)])

    return pl.kernel(
        body,
        out_type=jax.ShapeDtypeStruct((_NC, _N, 144), jnp.float32),
        mesh=mesh,
        compiler_params=pltpu.CompilerParams(use_tc_tiling_on_sc=False),
        scratch_types=(
            [pltpu.VMEM_SHARED((_N, 144), jnp.float32),
             pltpu.VMEM((_NCHP, _CP), jnp.int32),
             pltpu.VMEM((_NCHP, _CP), jnp.int32)]
            + [pltpu.VMEM((_CP, 144), jnp.float32) for _ in range(_RING)]
            + [pltpu.VMEM((_CP, 16), jnp.float32) for _ in range(_RING)]
            + [pltpu.SemaphoreType.DMA for _ in range(3 * _RING)]
        ),
    )


@functools.lru_cache(maxsize=None)
def _edge_pass(H):
    return _make_edge_pass(H)


def _edge_cat_body(src_hbm, dst_hbm, a_hbm, b_hbm, y1_out,
                   srcv, dstv,
                   ba0, ba1, ba2, ba3, ba4,
                   sa0, sa1, sa2, sa3, sa4,
                   sb0, sb1, sb2, sb3, sb4,
                   sw0, sw1, sw2, sw3, sw4):
    ba = [ba0, ba1, ba2, ba3, ba4]
    sa = [sa0, sa1, sa2, sa3, sa4]
    sb = [sb0, sb1, sb2, sb3, sb4]
    sw = [sw0, sw1, sw2, sw3, sw4]

    c = lax.axis_index("c")
    s = lax.axis_index("s")
    wid = s * _NC + c

    pltpu.sync_copy(src_hbm.at[wid], srcv)
    pltpu.sync_copy(dst_hbm.at[wid], dstv)

    def issue_a(j, b):
        pltpu.async_copy(a_hbm.at[srcv.at[j]], ba[b], sa[b])

    def wait_a(j, b):
        pltpu.make_async_copy(a_hbm.at[srcv.at[j]], ba[b], sa[b]).wait()

    def issue_b(j, b):
        pltpu.async_copy(b_hbm.at[dstv.at[j]], ba[b], sb[b], add=True)

    def wait_b(j, b):
        pltpu.make_async_copy(b_hbm.at[dstv.at[j]], ba[b], sb[b]).wait()

    def issue_w(j, b):
        pltpu.async_copy(ba[b], y1_out.at[pl.ds((wid * _NCH + j) * _C, _C)],
                         sw[b])

    def wait_w(j, b):
        pltpu.make_async_copy(ba[b],
                              y1_out.at[pl.ds((wid * _NCH + j) * _C, _C)],
                              sw[b]).wait()

    issue_a(0, 0)
    issue_a(1, 1)
    wait_a(0, 0)
    issue_b(0, 0)

    def outer(jo, carry):
        for b in range(_RING):
            j = jo * _RING + b

            @pl.when(j >= 2)
            def _(b=b, j=j):
                wait_w(j - 2, (b - 2) % _RING)

            @pl.when(j <= _NCH - 3)
            def _(b=b, j=j):
                issue_a(j + 2, (b + 2) % _RING)

            @pl.when(j <= _NCH - 2)
            def _(b=b, j=j):
                wait_a(j + 1, (b + 1) % _RING)
                issue_b(j + 1, (b + 1) % _RING)

            wait_b(j, b)
            issue_w(j, b)
        return 0

    lax.fori_loop(0, _OUTER, outer, 0)
    wait_w(_NCH - 2, (_NCH - 2) % _RING)
    wait_w(_NCH - 1, (_NCH - 1) % _RING)


@functools.lru_cache(maxsize=None)
def _edge_cat_kernel():
    return pl.kernel(
        _edge_cat_body,
        out_type=jax.ShapeDtypeStruct((_E, 128), jnp.float32),
        mesh=plsc.VectorSubcoreMesh(core_axis_name="c", subcore_axis_name="s",
                                    num_cores=_NC, num_subcores=_NS),
        scratch_types=(
            [pltpu.VMEM((_NCH, _C), jnp.int32),
             pltpu.VMEM((_NCH, _C), jnp.int32)]
            + [pltpu.VMEM((_C, 128), jnp.float32) for _ in range(_RING)]
            + [pltpu.SemaphoreType.DMA for _ in range(3 * _RING)]
        ),
    )


def _dup_table_mat(al, H):
    d = 128 // H
    k = np.arange(128)
    mask = np.equal(k[:, None] // d, np.arange(H)[None, :]).astype(np.float32)
    blkdiag = al.T[k % d, :] * mask
    return blkdiag[:, np.arange(16) % H]


def _expand_mat(H):
    d = 128 // H
    P = np.zeros((16, 128), np.float32)
    for f in range(128):
        P[f // d, f] = 1.0
    return jnp.asarray(P)


def kernel(h, e, edge_index, params):
    p = params
    srcp = edge_index[0].reshape(_TILES, _NCHP, _CP)
    dstp = edge_index[1].reshape(_TILES, _NCHP, _CP)
    src2 = edge_index[0].reshape(_TILES, _NCH, _C)
    dst2 = edge_index[1].reshape(_TILES, _NCH, _C)
    z144 = jnp.zeros((_N, 144), jnp.float32)

    x = _embed(h, p["W_emb"], p["b_emb"])

    layers = [(p["W0"], p["al0"], p["ar0"], 8),
              (p["W1"], p["al1"], p["ar1"], 8),
              (p["W2"], p["al2"], p["ar2"], 8),
              (p["W3"], p["al3"], p["ar3"], 1)]

    s_out = None
    for li, (W, al, ar, H) in enumerate(layers):
        Aal = _dup_table_mat(al, H)
        Aar = _dup_table_mat(ar, H)
        zel, erd = _pre(x, W, Aal, Aar)
        acc = _edge_pass(H)(srcp, dstp, zel, erd, z144)
        x = _fin(acc, x, _expand_mat(H))
        if li == 1:
            s_out = _assign(x, p["Wassign"])

    A, B = _ab(x, p["fcW0"][:128], p["fcW0"][128:], p["fcb0"])
    y1 = _edge_cat_kernel()(src2, dst2, A, B)
    y = _mlp(y1, p["fcW1"], p["fcb1"], p["fcW2"], p["fcb2"])
    return (y, s_out)


# speedup vs baseline: 81.2539x; 1.0240x over previous
import functools

import numpy as np
import jax
import jax.numpy as jnp
from jax import lax
from jax.experimental import pallas as pl
from jax.experimental.pallas import tpu as pltpu
from jax.experimental.pallas import tpu_sc as plsc

_N = 10000
_E = 320000
_NBLK = 10
_BN = _N // _NBLK

_NC = 2
_NS = 16
_TILES = _NC * _NS
_EPT = _E // _TILES
_C = 80
_KB = 25
_NKB = _EPT // (_C * _KB)
_RPT = _EPT // _C
_NR = _N // _NS

_EBLK = 40
_BE = _E // _EBLK


def _zel_er(z, al_ref, ar_ref):
    el = jnp.dot(z, al_ref[...], preferred_element_type=jnp.float32)
    er = jnp.dot(z, ar_ref[...], preferred_element_type=jnp.float32)
    return jnp.concatenate([z, el], axis=1), er


def _emb_pre_body(h_ref, we_ref, be_ref, w_ref, al_ref, ar_ref,
                  x_ref, zel_ref, er_ref):
    x = jnp.dot(h_ref[...], we_ref[...],
                preferred_element_type=jnp.float32) + be_ref[...]
    x_ref[...] = x
    z = jnp.dot(x, w_ref[...], preferred_element_type=jnp.float32)
    zel_ref[...], er_ref[...] = _zel_er(z, al_ref, ar_ref)


def _emb_pre(h, We, be, W, Aal, Aar):
    return pl.pallas_call(
        _emb_pre_body,
        grid=(_NBLK,),
        in_specs=[pl.BlockSpec((_BN, 128), lambda i: (i, 0)),
                  pl.BlockSpec((128, 128), lambda i: (0, 0)),
                  pl.BlockSpec((1, 128), lambda i: (0, 0)),
                  pl.BlockSpec((128, 128), lambda i: (0, 0)),
                  pl.BlockSpec((128, 16), lambda i: (0, 0)),
                  pl.BlockSpec((128, 16), lambda i: (0, 0))],
        out_specs=[pl.BlockSpec((_BN, 128), lambda i: (i, 0)),
                   pl.BlockSpec((_BN, 144), lambda i: (i, 0)),
                   pl.BlockSpec((_BN, 16), lambda i: (i, 0))],
        out_shape=[jax.ShapeDtypeStruct((_N, 128), jnp.float32),
                   jax.ShapeDtypeStruct((_N, 144), jnp.float32),
                   jax.ShapeDtypeStruct((_N, 16), jnp.float32)],
    )(h, We, be.reshape(1, 128), W, Aal, Aar)


def _finalize(acc_ref, x_ref, p_ref):
    t = acc_ref[0] + acc_ref[1]
    rec = 1.0 / (t[:, 128:] + 1e-9)
    recx = jnp.dot(rec, p_ref[...], preferred_element_type=jnp.float32)
    o = t[:, :128] * recx
    o = jnp.where(o > 0, o, jnp.exp(jnp.minimum(o, 0.0)) - 1.0)
    xprev = x_ref[...]
    xprev = jnp.where(jnp.isinf(xprev), 1e9, xprev)
    return o + xprev


def _fin_pre_body(acc_ref, x_ref, p_ref, w_ref, al_ref, ar_ref,
                  xo_ref, zel_ref, er_ref):
    xn = _finalize(acc_ref, x_ref, p_ref)
    xo_ref[...] = xn
    xc = jnp.where(jnp.isinf(xn), 1e9, xn)
    z = jnp.dot(xc, w_ref[...], preferred_element_type=jnp.float32)
    zel_ref[...], er_ref[...] = _zel_er(z, al_ref, ar_ref)


def _fin_pre(acc, x, P, W, Aal, Aar):
    return pl.pallas_call(
        _fin_pre_body,
        grid=(_NBLK,),
        in_specs=[pl.BlockSpec((_NC, _BN, 144), lambda i: (0, i, 0)),
                  pl.BlockSpec((_BN, 128), lambda i: (i, 0)),
                  pl.BlockSpec((16, 128), lambda i: (0, 0)),
                  pl.BlockSpec((128, 128), lambda i: (0, 0)),
                  pl.BlockSpec((128, 16), lambda i: (0, 0)),
                  pl.BlockSpec((128, 16), lambda i: (0, 0))],
        out_specs=[pl.BlockSpec((_BN, 128), lambda i: (i, 0)),
                   pl.BlockSpec((_BN, 144), lambda i: (i, 0)),
                   pl.BlockSpec((_BN, 16), lambda i: (i, 0))],
        out_shape=[jax.ShapeDtypeStruct((_N, 128), jnp.float32),
                   jax.ShapeDtypeStruct((_N, 144), jnp.float32),
                   jax.ShapeDtypeStruct((_N, 16), jnp.float32)],
    )(acc, x, P, W, Aal, Aar)


def _fin_pre_assign_body(acc_ref, x_ref, p_ref, w_ref, al_ref, ar_ref,
                         wassign_ref, xo_ref, zel_ref, er_ref, s_ref):
    xn = _finalize(acc_ref, x_ref, p_ref)
    xo_ref[...] = xn
    l = jnp.dot(xn, wassign_ref[...], preferred_element_type=jnp.float32)
    m = jnp.max(l, axis=1, keepdims=True)
    exl = jnp.exp(l - m)
    s_ref[...] = exl / jnp.sum(exl, axis=1, keepdims=True)
    xc = jnp.where(jnp.isinf(xn), 1e9, xn)
    z = jnp.dot(xc, w_ref[...], preferred_element_type=jnp.float32)
    zel_ref[...], er_ref[...] = _zel_er(z, al_ref, ar_ref)


def _fin_pre_assign(acc, x, P, W, Aal, Aar, Wassign):
    return pl.pallas_call(
        _fin_pre_assign_body,
        grid=(_NBLK,),
        in_specs=[pl.BlockSpec((_NC, _BN, 144), lambda i: (0, i, 0)),
                  pl.BlockSpec((_BN, 128), lambda i: (i, 0)),
                  pl.BlockSpec((16, 128), lambda i: (0, 0)),
                  pl.BlockSpec((128, 128), lambda i: (0, 0)),
                  pl.BlockSpec((128, 16), lambda i: (0, 0)),
                  pl.BlockSpec((128, 16), lambda i: (0, 0)),
                  pl.BlockSpec((128, 100), lambda i: (0, 0))],
        out_specs=[pl.BlockSpec((_BN, 128), lambda i: (i, 0)),
                   pl.BlockSpec((_BN, 144), lambda i: (i, 0)),
                   pl.BlockSpec((_BN, 16), lambda i: (i, 0)),
                   pl.BlockSpec((_BN, 100), lambda i: (i, 0))],
        out_shape=[jax.ShapeDtypeStruct((_N, 128), jnp.float32),
                   jax.ShapeDtypeStruct((_N, 144), jnp.float32),
                   jax.ShapeDtypeStruct((_N, 16), jnp.float32),
                   jax.ShapeDtypeStruct((_N, 100), jnp.float32)],
    )(acc, x, P, W, Aal, Aar, Wassign)


def _fin_ab_body(acc_ref, x_ref, p_ref, wa_ref, wb_ref, b_ref,
                 a_ref, bo_ref):
    xn = _finalize(acc_ref, x_ref, p_ref)
    xc = jnp.where(jnp.isinf(xn), 1e9, xn)
    a_ref[...] = jnp.dot(xc, wa_ref[...],
                         preferred_element_type=jnp.float32) + b_ref[...]
    bo_ref[...] = jnp.dot(xc, wb_ref[...], preferred_element_type=jnp.float32)


def _fin_ab(acc, x, P, Wa, Wb, b0):
    return pl.pallas_call(
        _fin_ab_body,
        grid=(_NBLK,),
        in_specs=[pl.BlockSpec((_NC, _BN, 144), lambda i: (0, i, 0)),
                  pl.BlockSpec((_BN, 128), lambda i: (i, 0)),
                  pl.BlockSpec((16, 128), lambda i: (0, 0)),
                  pl.BlockSpec((128, 128), lambda i: (0, 0)),
                  pl.BlockSpec((128, 128), lambda i: (0, 0)),
                  pl.BlockSpec((1, 128), lambda i: (0, 0))],
        out_specs=[pl.BlockSpec((_BN, 128), lambda i: (i, 0)),
                   pl.BlockSpec((_BN, 128), lambda i: (i, 0))],
        out_shape=[jax.ShapeDtypeStruct((_N, 128), jnp.float32),
                   jax.ShapeDtypeStruct((_N, 128), jnp.float32)],
    )(acc, x, P, Wa, Wb, b0.reshape(1, 128))


def _mlp_body(y1_ref, w1_ref, b1_ref, w2_ref, b2_ref, o_ref):
    y1 = jnp.maximum(y1_ref[...], 0.0)
    t = jnp.dot(y1, w1_ref[...],
                preferred_element_type=jnp.float32) + b1_ref[...]
    t = jnp.maximum(t, 0.0)
    y = jnp.dot(t, w2_ref[...], preferred_element_type=jnp.float32) + b2_ref[...]
    o_ref[...] = jnp.where(jnp.isinf(y), 1e9, y)


def _mlp(y1, W1, b1, W2, b2):
    return pl.pallas_call(
        _mlp_body,
        grid=(_EBLK,),
        in_specs=[pl.BlockSpec((_BE, 128), lambda i: (i, 0)),
                  pl.BlockSpec((128, 64), lambda i: (0, 0)),
                  pl.BlockSpec((1, 64), lambda i: (0, 0)),
                  pl.BlockSpec((64, 2), lambda i: (0, 0)),
                  pl.BlockSpec((1, 2), lambda i: (0, 0))],
        out_specs=pl.BlockSpec((_BE, 2), lambda i: (i, 0)),
        out_shape=jax.ShapeDtypeStruct((_E, 2), jnp.float32),
    )(y1, W1, b1.reshape(1, 64), W2, b2.reshape(1, 2))


_RING = 5
_CP = 20
_NCHP = _EPT // _CP
_OUTERP = _NCHP // _RING
_NCH = _EPT // _C
_OUTER = _NCH // _RING


def _make_edge_pass(H):
    cols = [(g * H) // 8 for g in range(8)]
    mesh = plsc.VectorSubcoreMesh(core_axis_name="c", subcore_axis_name="s",
                                  num_cores=_NC, num_subcores=_NS)

    def body(src_hbm, dst_hbm, zel_hbm, er_hbm, zero_hbm, acc_out,
             sacc, srcv, dstv,
             zsr0, zsr1, zsr2, zsr3, zsr4,
             err0, err1, err2, err3, err4,
             sg0, sg1, sg2, sg3, sg4,
             se0, se1, se2, se3, se4,
             ss0, ss1, ss2, ss3, ss4):
        zsr = [zsr0, zsr1, zsr2, zsr3, zsr4]
        errb = [err0, err1, err2, err3, err4]
        sg = [sg0, sg1, sg2, sg3, sg4]
        se = [se0, se1, se2, se3, se4]
        ss = [ss0, ss1, ss2, ss3, ss4]

        c = lax.axis_index("c")
        s = lax.axis_index("s")
        wid = s * _NC + c

        @pl.when(s < 15)
        def _():
            pltpu.sync_copy(zero_hbm.at[pl.ds(s * 624, 624)],
                            sacc.at[pl.ds(s * 624, 624)])

        @pl.when(s == 15)
        def _():
            pltpu.sync_copy(zero_hbm.at[pl.ds(9360, 640)],
                            sacc.at[pl.ds(9360, 640)])

        pltpu.sync_copy(src_hbm.at[wid], srcv)
        pltpu.sync_copy(dst_hbm.at[wid], dstv)
        plsc.subcore_barrier()

        def issue_g(j, b):
            pltpu.async_copy(zel_hbm.at[srcv.at[j]], zsr[b], sg[b])
            pltpu.async_copy(er_hbm.at[dstv.at[j]], errb[b], se[b])

        def wait_g(j, b):
            pltpu.make_async_copy(zel_hbm.at[srcv.at[j]], zsr[b], sg[b]).wait()
            pltpu.make_async_copy(er_hbm.at[dstv.at[j]], errb[b], se[b]).wait()

        def issue_s(j, b):
            pltpu.async_copy(zsr[b], sacc.at[dstv.at[j]], ss[b], add=True)

        def wait_s(j, b):
            pltpu.make_async_copy(zsr[b], sacc.at[dstv.at[j]], ss[b]).wait()

        issue_g(0, 0)
        issue_g(1, 1)

        def outer(jo, carry):
            for b in range(_RING):
                j = jo * _RING + b
                wait_g(j, b)
                zb = zsr[b]
                eb = errb[b]

                @plsc.parallel_loop(0, _CP, 1, unroll=4)
                def _(i, zb=zb, eb=eb):
                    lg = zb[i, pl.ds(128, 16)] + eb[i]
                    lg = jnp.maximum(lg, 0.2 * lg)
                    ex = jnp.exp(lg)
                    zb[i, pl.ds(128, 16)] = ex
                    for g in range(8):
                        mult = ex[jnp.full((16,), cols[g], jnp.int32)]
                        sl = pl.ds(g * 16, 16)
                        zb[i, sl] = zb[i, sl] * mult
                issue_s(j, b)

                @pl.when(j >= 2)
                def _(b=b, j=j):
                    wait_s(j - 2, (b - 2) % _RING)

                @pl.when(j <= _NCHP - 3)
                def _(b=b, j=j):
                    issue_g(j + 2, (b + 2) % _RING)
            return 0

        lax.fori_loop(0, _OUTERP, outer, 0)
        wait_s(_NCHP - 2, (_NCHP - 2) % _RING)
        wait_s(_NCHP - 1, (_NCHP - 1) % _RING)

        plsc.subcore_barrier()

        @pl.when(s < 15)
        def _():
            pltpu.sync_copy(sacc.at[pl.ds(s * 624, 624)],
                            acc_out.at[c, pl.ds(s * 624, 624)])

        @pl.when(s == 15)
        def _():
            pltpu.sync_copy(sacc.at[pl.ds(9360, 640)],
                            acc_out.at[c, pl.ds(9360, 640)])

    return pl.kernel(
        body,
        out_type=jax.ShapeDtypeStruct((_NC, _N, 144), jnp.float32),
        mesh=mesh,
        compiler_params=pltpu.CompilerParams(use_tc_tiling_on_sc=False),
        scratch_types=(
            [pltpu.VMEM_SHARED((_N, 144), jnp.float32),
             pltpu.VMEM((_NCHP, _CP), jnp.int32),
             pltpu.VMEM((_NCHP, _CP), jnp.int32)]
            + [pltpu.VMEM((_CP, 144), jnp.float32) for _ in range(_RING)]
            + [pltpu.VMEM((_CP, 16), jnp.float32) for _ in range(_RING)]
            + [pltpu.SemaphoreType.DMA for _ in range(3 * _RING)]
        ),
    )


@functools.lru_cache(maxsize=None)
def _edge_pass(H):
    return _make_edge_pass(H)


def _edge_cat_body(src_hbm, dst_hbm, a_hbm, b_hbm, y1_out,
                   srcv, dstv,
                   ba0, ba1, ba2, ba3, ba4,
                   sa0, sa1, sa2, sa3, sa4,
                   sb0, sb1, sb2, sb3, sb4,
                   sw0, sw1, sw2, sw3, sw4):
    ba = [ba0, ba1, ba2, ba3, ba4]
    sa = [sa0, sa1, sa2, sa3, sa4]
    sb = [sb0, sb1, sb2, sb3, sb4]
    sw = [sw0, sw1, sw2, sw3, sw4]

    c = lax.axis_index("c")
    s = lax.axis_index("s")
    wid = s * _NC + c

    pltpu.sync_copy(src_hbm.at[wid], srcv)
    pltpu.sync_copy(dst_hbm.at[wid], dstv)

    def issue_a(j, b):
        pltpu.async_copy(a_hbm.at[srcv.at[j]], ba[b], sa[b])

    def wait_a(j, b):
        pltpu.make_async_copy(a_hbm.at[srcv.at[j]], ba[b], sa[b]).wait()

    def issue_b(j, b):
        pltpu.async_copy(b_hbm.at[dstv.at[j]], ba[b], sb[b], add=True)

    def wait_b(j, b):
        pltpu.make_async_copy(b_hbm.at[dstv.at[j]], ba[b], sb[b]).wait()

    def issue_w(j, b):
        pltpu.async_copy(ba[b], y1_out.at[pl.ds((wid * _NCH + j) * _C, _C)],
                         sw[b])

    def wait_w(j, b):
        pltpu.make_async_copy(ba[b],
                              y1_out.at[pl.ds((wid * _NCH + j) * _C, _C)],
                              sw[b]).wait()

    issue_a(0, 0)
    issue_a(1, 1)
    wait_a(0, 0)
    issue_b(0, 0)

    def outer(jo, carry):
        for b in range(_RING):
            j = jo * _RING + b

            @pl.when(j >= 2)
            def _(b=b, j=j):
                wait_w(j - 2, (b - 2) % _RING)

            @pl.when(j <= _NCH - 3)
            def _(b=b, j=j):
                issue_a(j + 2, (b + 2) % _RING)

            @pl.when(j <= _NCH - 2)
            def _(b=b, j=j):
                wait_a(j + 1, (b + 1) % _RING)
                issue_b(j + 1, (b + 1) % _RING)

            wait_b(j, b)
            issue_w(j, b)
        return 0

    lax.fori_loop(0, _OUTER, outer, 0)
    wait_w(_NCH - 2, (_NCH - 2) % _RING)
    wait_w(_NCH - 1, (_NCH - 1) % _RING)


@functools.lru_cache(maxsize=None)
def _edge_cat_kernel():
    return pl.kernel(
        _edge_cat_body,
        out_type=jax.ShapeDtypeStruct((_E, 128), jnp.float32),
        mesh=plsc.VectorSubcoreMesh(core_axis_name="c", subcore_axis_name="s",
                                    num_cores=_NC, num_subcores=_NS),
        scratch_types=(
            [pltpu.VMEM((_NCH, _C), jnp.int32),
             pltpu.VMEM((_NCH, _C), jnp.int32)]
            + [pltpu.VMEM((_C, 128), jnp.float32) for _ in range(_RING)]
            + [pltpu.SemaphoreType.DMA for _ in range(3 * _RING)]
        ),
    )


def _dup_table_mat(al, H):
    d = 128 // H
    k = np.arange(128)
    mask = np.equal(k[:, None] // d, np.arange(H)[None, :]).astype(np.float32)
    blkdiag = al.T[k % d, :] * mask
    return blkdiag[:, np.arange(16) % H]


def _expand_mat(H):
    d = 128 // H
    P = np.zeros((16, 128), np.float32)
    for f in range(128):
        P[f // d, f] = 1.0
    return jnp.asarray(P)


def kernel(h, e, edge_index, params):
    p = params
    srcp = edge_index[0].reshape(_TILES, _NCHP, _CP)
    dstp = edge_index[1].reshape(_TILES, _NCHP, _CP)
    src2 = edge_index[0].reshape(_TILES, _NCH, _C)
    dst2 = edge_index[1].reshape(_TILES, _NCH, _C)
    z144 = jnp.zeros((_N, 144), jnp.float32)

    P8 = _expand_mat(8)
    P1 = _expand_mat(1)
    ep8 = _edge_pass(8)
    ep1 = _edge_pass(1)

    x, zel, erd = _emb_pre(h, p["W_emb"], p["b_emb"], p["W0"],
                           _dup_table_mat(p["al0"], 8),
                           _dup_table_mat(p["ar0"], 8))
    acc = ep8(srcp, dstp, zel, erd, z144)
    x, zel, erd = _fin_pre(acc, x, P8, p["W1"],
                           _dup_table_mat(p["al1"], 8),
                           _dup_table_mat(p["ar1"], 8))
    acc = ep8(srcp, dstp, zel, erd, z144)
    x, zel, erd, s_out = _fin_pre_assign(acc, x, P8, p["W2"],
                                         _dup_table_mat(p["al2"], 8),
                                         _dup_table_mat(p["ar2"], 8),
                                         p["Wassign"])
    acc = ep8(srcp, dstp, zel, erd, z144)
    x, zel, erd = _fin_pre(acc, x, P8, p["W3"],
                           _dup_table_mat(p["al3"], 1),
                           _dup_table_mat(p["ar3"], 1))
    acc = ep1(srcp, dstp, zel, erd, z144)
    A, B = _fin_ab(acc, x, P1, p["fcW0"][:128], p["fcW0"][128:], p["fcb0"])
    y1 = _edge_cat_kernel()(src2, dst2, A, B)
    y = _mlp(y1, p["fcW1"], p["fcb1"], p["fcW2"], p["fcb2"])
    return (y, s_out)
